# Initial kernel scaffold; baseline (speedup 1.0000x reference)
import jax
import jax.numpy as jnp
from jax.experimental import pallas as pl


def kernel(x, edge_index, W1, b1, W2, b2):
    raise NotImplementedError("write your pallas kernel here")


# speedup vs baseline: 10.5027x; 10.5027x over previous
import functools

import jax
import jax.numpy as jnp
from jax import lax
from jax.experimental import pallas as pl
from jax.experimental.pallas import tpu as pltpu
from jax.experimental.pallas import tpu_sc as plsc

NC = 2
NS = 16
NT = NC * NS
CHUNK = 128
IB = 16


def _ru(a: int, b: int) -> int:
    return (a + b - 1) // b * b


def _make_deg_kernel(n_pad: int, cpt: int):
    rpt = n_pad // NS
    mesh = plsc.VectorSubcoreMesh(core_axis_name="c", subcore_axis_name="s")

    @functools.partial(
        pl.kernel,
        out_type=jax.ShapeDtypeStruct((NC, n_pad, 16), jnp.float32),
        mesh=mesh,
        compiler_params=pltpu.CompilerParams(use_tc_tiling_on_sc=False),
        scratch_types=[
            pltpu.VMEM((cpt, CHUNK), jnp.int32),
            pltpu.VMEM((CHUNK, 16), jnp.float32),
            pltpu.VMEM((rpt, 16), jnp.float32),
            pltpu.VMEM_SHARED((n_pad, 16), jnp.float32),
        ],
    )
    def deg_kernel(dst_hbm, deg_out, dst_v, ones_v, zeros_v, deg_sh):
        cid = lax.axis_index("c")
        sid = lax.axis_index("s")
        wid = cid * NS + sid
        pltpu.sync_copy(dst_hbm.at[pl.ds(wid * cpt, cpt)], dst_v)

        def fill_ones(i, _):
            ones_v[i] = jnp.full((16,), 1.0, jnp.float32)
            return 0

        lax.fori_loop(0, CHUNK, fill_ones, 0)

        def fill_zeros(i, _):
            zeros_v[i] = jnp.zeros((16,), jnp.float32)
            return 0

        lax.fori_loop(0, rpt, fill_zeros, 0)

        pltpu.sync_copy(zeros_v, deg_sh.at[pl.ds(sid * rpt, rpt)])
        plsc.subcore_barrier()

        def body(j, _):
            pltpu.sync_copy(ones_v, deg_sh.at[dst_v.at[j]], add=True)
            return 0

        lax.fori_loop(0, cpt, body, 0)
        plsc.subcore_barrier()
        pltpu.sync_copy(
            deg_sh.at[pl.ds(sid * rpt, rpt)],
            deg_out.at[cid].at[pl.ds(sid * rpt, rpt)],
        )

    return deg_kernel


def _make_agg_kernel(n_pad: int, d: int, cpt: int, tc_tiling: bool):
    rpt = n_pad // NS
    nblk = cpt // IB
    mesh = plsc.VectorSubcoreMesh(core_axis_name="c", subcore_axis_name="s")

    @functools.partial(
        pl.kernel,
        out_type=jax.ShapeDtypeStruct((NC, n_pad, d), jnp.float32),
        mesh=mesh,
        compiler_params=pltpu.CompilerParams(use_tc_tiling_on_sc=tc_tiling),
        scratch_types=[
            pltpu.VMEM((IB, CHUNK), jnp.int32),
            pltpu.VMEM((IB, CHUNK), jnp.int32),
            pltpu.VMEM((CHUNK, d), jnp.float32),
            pltpu.VMEM((CHUNK, d), jnp.float32),
            pltpu.VMEM_SHARED((n_pad, d), jnp.float32),
            pltpu.SemaphoreType.DMA,
            pltpu.SemaphoreType.DMA,
        ],
    )
    def agg_kernel(y_hbm, src_hbm, dst_hbm, z_out,
                   src_v, dst_v, rows0, rows1, z_sh, sem0, sem1):
        cid = lax.axis_index("c")
        sid = lax.axis_index("s")
        wid = cid * NS + sid

        pltpu.sync_copy(
            y_hbm.at[pl.ds(sid * rpt, rpt)],
            z_sh.at[pl.ds(sid * rpt, rpt)],
        )
        plsc.subcore_barrier()

        def outer(t, _):
            base = wid * cpt + t * IB
            pltpu.sync_copy(src_hbm.at[pl.ds(base, IB)], src_v)
            pltpu.sync_copy(dst_hbm.at[pl.ds(base, IB)], dst_v)

            def body(j2, _):
                j = j2 * 2
                g0 = pltpu.async_copy(y_hbm.at[src_v.at[j]], rows0, sem0)
                g1 = pltpu.async_copy(y_hbm.at[src_v.at[j + 1]], rows1, sem1)
                g0.wait()
                pltpu.sync_copy(rows0, z_sh.at[dst_v.at[j]], add=True)
                g1.wait()
                pltpu.sync_copy(rows1, z_sh.at[dst_v.at[j + 1]], add=True)
                return 0

            lax.fori_loop(0, IB // 2, body, 0)
            return 0

        lax.fori_loop(0, nblk, outer, 0)
        plsc.subcore_barrier()
        pltpu.sync_copy(
            z_sh.at[pl.ds(sid * rpt, rpt)],
            z_out.at[cid].at[pl.ds(sid * rpt, rpt)],
        )

    return agg_kernel


def _tc1_body(x_ref, w_ref, d0_ref, d1_ref, y_ref, dinv_ref):
    deg = d0_ref[...] + d1_ref[...] + 1.0
    dinv = lax.rsqrt(deg)
    dinv_ref[...] = dinv
    y_ref[...] = jnp.dot(x_ref[...], w_ref[...],
                         preferred_element_type=jnp.float32) * dinv[:, None]


def _tc2_body(z0_ref, z1_ref, y1_ref, dinv_ref, b1_ref, w2_ref,
              emb_ref, y2_ref):
    dinv = dinv_ref[...]
    emb = (z0_ref[...] + z1_ref[...] - y1_ref[...]) * dinv[:, None] \
        + b1_ref[...][None, :]
    emb_ref[...] = emb
    h = jnp.maximum(emb, 0.0)
    y2_ref[...] = jnp.dot(h, w2_ref[...],
                          preferred_element_type=jnp.float32) * dinv[:, None]


def _tc3_body(c_real, z0_ref, z1_ref, y2_ref, dinv_ref, b2_ref, out_ref):
    dinv = dinv_ref[...]
    t = (z0_ref[...] + z1_ref[...] - y2_ref[...]) * dinv[:, None] \
        + b2_ref[...][None, :]
    col = lax.broadcasted_iota(jnp.int32, t.shape, 1)
    t = jnp.where(col < c_real, t, -jnp.inf)
    m = jnp.max(t, axis=1, keepdims=True)
    s = jnp.sum(jnp.exp(t - m), axis=1, keepdims=True)
    out_ref[...] = t - m - jnp.log(s)


def kernel(x, edge_index, W1, b1, W2, b2):
    n, f_in = x.shape
    hid = W1.shape[1]
    c = W2.shape[1]
    e = edge_index.shape[1]

    br = 512
    n_pad = _ru(n + 1, br)
    assert n_pad % (NS * 8) == 0
    ept = _ru(-(-e // NT), IB * CHUNK)
    cpt = ept // CHUNK
    e_pad = ept * NT
    d2 = _ru(c, 64)

    pad_e = e_pad - e
    src = jnp.concatenate(
        [edge_index[0], jnp.full((pad_e,), n, jnp.int32)]).reshape(NT * cpt, CHUNK)
    dst = jnp.concatenate(
        [edge_index[1], jnp.full((pad_e,), n, jnp.int32)]).reshape(NT * cpt, CHUNK)
    x_pad = jnp.pad(x, ((0, n_pad - n), (0, 0)))
    w2p = jnp.pad(W2, ((0, 0), (0, d2 - c)))
    b2p = jnp.pad(b2, (0, d2 - c))

    grid = n_pad // br
    row_spec = pl.BlockSpec((br, hid), lambda i: (i, 0))
    row_spec2 = pl.BlockSpec((br, d2), lambda i: (i, 0))
    vec_spec = pl.BlockSpec((br,), lambda i: (i,))

    deg3 = _make_deg_kernel(n_pad, cpt)(dst)
    dp0 = deg3[0, :, 0]
    dp1 = deg3[1, :, 0]

    y1, dinv = pl.pallas_call(
        _tc1_body,
        grid=(grid,),
        in_specs=[
            pl.BlockSpec((br, f_in), lambda i: (i, 0)),
            pl.BlockSpec((f_in, hid), lambda i: (0, 0)),
            vec_spec, vec_spec,
        ],
        out_specs=[row_spec, vec_spec],
        out_shape=[
            jax.ShapeDtypeStruct((n_pad, hid), jnp.float32),
            jax.ShapeDtypeStruct((n_pad,), jnp.float32),
        ],
    )(x_pad, W1, dp0, dp1)

    z1 = _make_agg_kernel(n_pad, hid, cpt, True)(y1, src, dst)

    emb, y2 = pl.pallas_call(
        _tc2_body,
        grid=(grid,),
        in_specs=[
            row_spec, row_spec, row_spec, vec_spec,
            pl.BlockSpec((hid,), lambda i: (0,)),
            pl.BlockSpec((hid, d2), lambda i: (0, 0)),
        ],
        out_specs=[row_spec, row_spec2],
        out_shape=[
            jax.ShapeDtypeStruct((n_pad, hid), jnp.float32),
            jax.ShapeDtypeStruct((n_pad, d2), jnp.float32),
        ],
    )(z1[0], z1[1], y1, dinv, b1, w2p)

    z2 = _make_agg_kernel(n_pad, d2, cpt, False)(y2, src, dst)

    out = pl.pallas_call(
        functools.partial(_tc3_body, c),
        grid=(grid,),
        in_specs=[
            row_spec2, row_spec2, row_spec2, vec_spec,
            pl.BlockSpec((d2,), lambda i: (0,)),
        ],
        out_specs=row_spec2,
        out_shape=jax.ShapeDtypeStruct((n_pad, d2), jnp.float32),
    )(z2[0], z2[1], y2, dinv, b2p)

    return (out[:n, :c], emb[:n])


# speedup vs baseline: 25.3052x; 2.4094x over previous
import functools

import jax
import jax.numpy as jnp
from jax import lax
from jax.experimental import pallas as pl
from jax.experimental.pallas import tpu as pltpu
from jax.experimental.pallas import tpu_sc as plsc

NC = 2
NS = 16
NT = NC * NS
CHUNK = 128
IB = 16


def _ru(a: int, b: int) -> int:
    return (a + b - 1) // b * b


def _make_deg_kernel(n_pad: int, cpt: int):
    rpt = n_pad // NS
    mesh = plsc.VectorSubcoreMesh(core_axis_name="c", subcore_axis_name="s")

    @functools.partial(
        pl.kernel,
        out_type=jax.ShapeDtypeStruct((NC, n_pad, 16), jnp.float32),
        mesh=mesh,
        compiler_params=pltpu.CompilerParams(use_tc_tiling_on_sc=False),
        scratch_types=[
            pltpu.VMEM((cpt, CHUNK), jnp.int32),
            pltpu.VMEM((CHUNK, 16), jnp.float32),
            pltpu.VMEM((rpt, 16), jnp.float32),
            pltpu.VMEM_SHARED((n_pad, 16), jnp.float32),
        ],
    )
    def deg_kernel(dst_hbm, deg_out, dst_v, ones_v, zeros_v, deg_sh):
        cid = lax.axis_index("c")
        sid = lax.axis_index("s")
        wid = cid * NS + sid
        pltpu.sync_copy(dst_hbm.at[pl.ds(wid * cpt, cpt)], dst_v)

        def fill_ones(i, _):
            ones_v[i] = jnp.full((16,), 1.0, jnp.float32)
            return 0

        lax.fori_loop(0, CHUNK, fill_ones, 0)

        def fill_zeros(i, _):
            zeros_v[i] = jnp.zeros((16,), jnp.float32)
            return 0

        lax.fori_loop(0, rpt, fill_zeros, 0)

        pltpu.sync_copy(zeros_v, deg_sh.at[pl.ds(sid * rpt, rpt)])
        plsc.subcore_barrier()

        def body(j, _):
            pltpu.sync_copy(ones_v, deg_sh.at[dst_v.at[j]], add=True)
            return 0

        lax.fori_loop(0, cpt, body, 0)
        plsc.subcore_barrier()
        pltpu.sync_copy(
            deg_sh.at[pl.ds(sid * rpt, rpt)],
            deg_out.at[cid].at[pl.ds(sid * rpt, rpt)],
        )

    return deg_kernel


def _make_agg_kernel(n_pad: int, d: int, cpt: int, tc_tiling: bool):
    rpt = n_pad // NS
    nblk = cpt // IB
    mesh = plsc.VectorSubcoreMesh(core_axis_name="c", subcore_axis_name="s")

    @functools.partial(
        pl.kernel,
        out_type=jax.ShapeDtypeStruct((NC, n_pad, d), jnp.float32),
        mesh=mesh,
        compiler_params=pltpu.CompilerParams(use_tc_tiling_on_sc=tc_tiling),
        scratch_types=[
            pltpu.VMEM((IB, CHUNK), jnp.int32),
            pltpu.VMEM((IB, CHUNK), jnp.int32),
            pltpu.VMEM((CHUNK, d), jnp.float32),
            pltpu.VMEM((CHUNK, d), jnp.float32),
            pltpu.VMEM_SHARED((n_pad, d), jnp.float32),
            pltpu.SemaphoreType.DMA,
            pltpu.SemaphoreType.DMA,
        ],
    )
    def agg_kernel(y_hbm, src_hbm, dst_hbm, z_out,
                   src_v, dst_v, rows0, rows1, z_sh, sem0, sem1):
        cid = lax.axis_index("c")
        sid = lax.axis_index("s")
        wid = cid * NS + sid

        pltpu.sync_copy(
            y_hbm.at[pl.ds(sid * rpt, rpt)],
            z_sh.at[pl.ds(sid * rpt, rpt)],
        )
        plsc.subcore_barrier()

        def outer(t, _):
            base = wid * cpt + t * IB
            pltpu.sync_copy(src_hbm.at[pl.ds(base, IB)], src_v)
            pltpu.sync_copy(dst_hbm.at[pl.ds(base, IB)], dst_v)

            def body(j2, _):
                j = j2 * 2
                g0 = pltpu.async_copy(y_hbm.at[src_v.at[j]], rows0, sem0)
                g1 = pltpu.async_copy(y_hbm.at[src_v.at[j + 1]], rows1, sem1)
                g0.wait()
                pltpu.sync_copy(rows0, z_sh.at[dst_v.at[j]], add=True)
                g1.wait()
                pltpu.sync_copy(rows1, z_sh.at[dst_v.at[j + 1]], add=True)
                return 0

            lax.fori_loop(0, IB // 2, body, 0)
            return 0

        lax.fori_loop(0, nblk, outer, 0)
        plsc.subcore_barrier()
        pltpu.sync_copy(
            z_sh.at[pl.ds(sid * rpt, rpt)],
            z_out.at[cid].at[pl.ds(sid * rpt, rpt)],
        )

    return agg_kernel


def _tc1_body(x_ref, w_ref, d0_ref, d1_ref, y_ref, dinv_ref):
    deg = d0_ref[...] + d1_ref[...] + 1.0
    dinv = lax.rsqrt(deg)
    dinv_ref[...] = dinv
    y_ref[...] = jnp.dot(x_ref[...], w_ref[...],
                         preferred_element_type=jnp.float32) * dinv[:, None]


def _tc2_body(z0_ref, z1_ref, y1_ref, dinv_ref, b1_ref, w2_ref,
              emb_ref, y2_ref):
    dinv = dinv_ref[...]
    emb = (z0_ref[...] + z1_ref[...] - y1_ref[...]) * dinv[:, None] \
        + b1_ref[...][None, :]
    emb_ref[...] = emb
    h = jnp.maximum(emb, 0.0)
    y2_ref[...] = jnp.dot(h, w2_ref[...],
                          preferred_element_type=jnp.float32) * dinv[:, None]


def _tc3_body(c_real, z0_ref, z1_ref, y2_ref, dinv_ref, b2_ref, out_ref):
    dinv = dinv_ref[...]
    t = (z0_ref[...] + z1_ref[...] - y2_ref[...]) * dinv[:, None] \
        + b2_ref[...][None, :]
    col = lax.broadcasted_iota(jnp.int32, t.shape, 1)
    t = jnp.where(col < c_real, t, -jnp.inf)
    m = jnp.max(t, axis=1, keepdims=True)
    s = jnp.sum(jnp.exp(t - m), axis=1, keepdims=True)
    out_ref[...] = t - m - jnp.log(s)


def kernel(x, edge_index, W1, b1, W2, b2):
    n, f_in = x.shape
    hid = W1.shape[1]
    c = W2.shape[1]
    e = edge_index.shape[1]

    br = 512
    n_pad = _ru(n + 1, br)
    assert n_pad % (NS * 8) == 0
    ept = _ru(-(-e // NT), IB * CHUNK)
    cpt = ept // CHUNK
    e_pad = ept * NT
    d2 = _ru(c, 64)

    pad_e = e_pad - e
    pad_idx = n + jnp.arange(pad_e, dtype=jnp.int32) % (n_pad - n)
    src = jnp.concatenate(
        [edge_index[0], pad_idx]).reshape(NT * cpt, CHUNK)
    dst = jnp.concatenate(
        [edge_index[1], pad_idx]).reshape(NT * cpt, CHUNK)
    x_pad = jnp.pad(x, ((0, n_pad - n), (0, 0)))
    w2p = jnp.pad(W2, ((0, 0), (0, d2 - c)))
    b2p = jnp.pad(b2, (0, d2 - c))

    grid = n_pad // br
    row_spec = pl.BlockSpec((br, hid), lambda i: (i, 0))
    row_spec2 = pl.BlockSpec((br, d2), lambda i: (i, 0))
    vec_spec = pl.BlockSpec((br,), lambda i: (i,))

    deg3 = _make_deg_kernel(n_pad, cpt)(dst)
    dp0 = deg3[0, :, 0]
    dp1 = deg3[1, :, 0]

    y1, dinv = pl.pallas_call(
        _tc1_body,
        grid=(grid,),
        in_specs=[
            pl.BlockSpec((br, f_in), lambda i: (i, 0)),
            pl.BlockSpec((f_in, hid), lambda i: (0, 0)),
            vec_spec, vec_spec,
        ],
        out_specs=[row_spec, vec_spec],
        out_shape=[
            jax.ShapeDtypeStruct((n_pad, hid), jnp.float32),
            jax.ShapeDtypeStruct((n_pad,), jnp.float32),
        ],
    )(x_pad, W1, dp0, dp1)

    z1 = _make_agg_kernel(n_pad, hid, cpt, True)(y1, src, dst)

    emb, y2 = pl.pallas_call(
        _tc2_body,
        grid=(grid,),
        in_specs=[
            row_spec, row_spec, row_spec, vec_spec,
            pl.BlockSpec((hid,), lambda i: (0,)),
            pl.BlockSpec((hid, d2), lambda i: (0, 0)),
        ],
        out_specs=[row_spec, row_spec2],
        out_shape=[
            jax.ShapeDtypeStruct((n_pad, hid), jnp.float32),
            jax.ShapeDtypeStruct((n_pad, d2), jnp.float32),
        ],
    )(z1[0], z1[1], y1, dinv, b1, w2p)

    z2 = _make_agg_kernel(n_pad, d2, cpt, False)(y2, src, dst)

    out = pl.pallas_call(
        functools.partial(_tc3_body, c),
        grid=(grid,),
        in_specs=[
            row_spec2, row_spec2, row_spec2, vec_spec,
            pl.BlockSpec((d2,), lambda i: (0,)),
        ],
        out_specs=row_spec2,
        out_shape=jax.ShapeDtypeStruct((n_pad, d2), jnp.float32),
    )(z2[0], z2[1], y2, dinv, b2p)

    return (out[:n, :c], emb[:n])


# speedup vs baseline: 26.3999x; 1.0433x over previous
import functools

import jax
import jax.numpy as jnp
from jax import lax
from jax.experimental import pallas as pl
from jax.experimental.pallas import tpu as pltpu
from jax.experimental.pallas import tpu_sc as plsc

NC = 2
NS = 16
NT = NC * NS
CHUNK = 128
IB = 16


def _ru(a: int, b: int) -> int:
    return (a + b - 1) // b * b


def _make_deg_kernel(n_pad: int, cpt: int):
    rpt = n_pad // NS
    mesh = plsc.VectorSubcoreMesh(core_axis_name="c", subcore_axis_name="s")

    @functools.partial(
        pl.kernel,
        out_type=jax.ShapeDtypeStruct((NC, n_pad, 16), jnp.float32),
        mesh=mesh,
        compiler_params=pltpu.CompilerParams(use_tc_tiling_on_sc=False),
        scratch_types=[
            pltpu.VMEM((cpt, CHUNK), jnp.int32),
            pltpu.VMEM((CHUNK, 16), jnp.float32),
            pltpu.VMEM((rpt, 16), jnp.float32),
            pltpu.VMEM_SHARED((n_pad, 16), jnp.float32),
        ],
    )
    def deg_kernel(dst_hbm, deg_out, dst_v, ones_v, zeros_v, deg_sh):
        cid = lax.axis_index("c")
        sid = lax.axis_index("s")
        wid = cid * NS + sid
        pltpu.sync_copy(dst_hbm.at[pl.ds(wid * cpt, cpt)], dst_v)

        def fill_ones(i, _):
            ones_v[i] = jnp.full((16,), 1.0, jnp.float32)
            return 0

        lax.fori_loop(0, CHUNK, fill_ones, 0)

        def fill_zeros(i, _):
            zeros_v[i] = jnp.zeros((16,), jnp.float32)
            return 0

        lax.fori_loop(0, rpt, fill_zeros, 0)

        pltpu.sync_copy(zeros_v, deg_sh.at[pl.ds(sid * rpt, rpt)])
        plsc.subcore_barrier()

        def body(j, _):
            pltpu.sync_copy(ones_v, deg_sh.at[dst_v.at[j]], add=True)
            return 0

        lax.fori_loop(0, cpt, body, 0)
        plsc.subcore_barrier()
        pltpu.sync_copy(
            deg_sh.at[pl.ds(sid * rpt, rpt)],
            deg_out.at[cid].at[pl.ds(sid * rpt, rpt)],
        )

    return deg_kernel


def _make_agg_kernel(n_pad: int, d: int, cpt: int, tc_tiling: bool):
    rpt = n_pad // NS
    nblk = cpt // IB
    mesh = plsc.VectorSubcoreMesh(core_axis_name="c", subcore_axis_name="s")

    ppb = IB // 2
    npairs = cpt // 2

    @functools.partial(
        pl.kernel,
        out_type=jax.ShapeDtypeStruct((NC, n_pad, d), jnp.float32),
        mesh=mesh,
        compiler_params=pltpu.CompilerParams(use_tc_tiling_on_sc=tc_tiling),
        scratch_types=[
            pltpu.VMEM((2, IB, CHUNK), jnp.int32),
            pltpu.VMEM((2, IB, CHUNK), jnp.int32),
            pltpu.VMEM((CHUNK, d), jnp.float32),
            pltpu.VMEM((CHUNK, d), jnp.float32),
            pltpu.VMEM_SHARED((n_pad, d), jnp.float32),
            pltpu.SemaphoreType.DMA,
            pltpu.SemaphoreType.DMA,
            pltpu.SemaphoreType.DMA,
            pltpu.SemaphoreType.DMA,
            pltpu.SemaphoreType.DMA,
        ],
    )
    def agg_kernel(y_hbm, src_hbm, dst_hbm, z_out,
                   src_v, dst_v, rows0, rows1, z_sh,
                   sg0, sg1, ss0, ss1, si):
        cid = lax.axis_index("c")
        sid = lax.axis_index("s")
        wid = cid * NS + sid

        pltpu.sync_copy(
            y_hbm.at[pl.ds(sid * rpt, rpt)],
            z_sh.at[pl.ds(sid * rpt, rpt)],
        )
        pltpu.sync_copy(src_hbm.at[pl.ds(wid * cpt, IB)], src_v.at[0])
        pltpu.sync_copy(dst_hbm.at[pl.ds(wid * cpt, IB)], dst_v.at[0])
        plsc.subcore_barrier()

        pltpu.async_copy(y_hbm.at[src_v.at[0, 0]], rows0, sg0)
        pltpu.async_copy(y_hbm.at[src_v.at[0, 1]], rows1, sg1)

        def body(p, _):
            t = p // ppb
            cur = lax.rem(t, 2)
            jj = lax.rem(p, ppb) * 2

            @pl.when(jnp.logical_and(jj == 0, t + 1 < nblk))
            def _():
                nxt = 1 - cur
                base = wid * cpt + (t + 1) * IB
                pltpu.async_copy(src_hbm.at[pl.ds(base, IB)],
                                 src_v.at[nxt], si)
                pltpu.async_copy(dst_hbm.at[pl.ds(base, IB)],
                                 dst_v.at[nxt], si)

            pltpu.make_async_copy(y_hbm.at[src_v.at[cur, jj]], rows0, sg0).wait()
            pltpu.async_copy(rows0, z_sh.at[dst_v.at[cur, jj]], ss0, add=True)
            pltpu.make_async_copy(y_hbm.at[src_v.at[cur, jj + 1]], rows1, sg1).wait()
            pltpu.async_copy(rows1, z_sh.at[dst_v.at[cur, jj + 1]], ss1, add=True)

            pn = jnp.minimum(p + 1, npairs - 1)
            tn = pn // ppb
            curn = lax.rem(tn, 2)
            jn = lax.rem(pn, ppb) * 2

            @pl.when(tn != t)
            def _():
                base = wid * cpt + tn * IB
                pltpu.make_async_copy(src_hbm.at[pl.ds(base, IB)],
                                      src_v.at[curn], si).wait()
                pltpu.make_async_copy(dst_hbm.at[pl.ds(base, IB)],
                                      dst_v.at[curn], si).wait()

            pltpu.make_async_copy(rows0, z_sh.at[dst_v.at[cur, jj]], ss0).wait()
            pltpu.async_copy(y_hbm.at[src_v.at[curn, jn]], rows0, sg0)
            pltpu.make_async_copy(rows1, z_sh.at[dst_v.at[cur, jj + 1]], ss1).wait()
            pltpu.async_copy(y_hbm.at[src_v.at[curn, jn + 1]], rows1, sg1)
            return 0

        lax.fori_loop(0, npairs, body, 0)
        last = npairs - 1
        tl = last // ppb
        curl = tl % 2
        jl = (last % ppb) * 2
        pltpu.make_async_copy(y_hbm.at[src_v.at[curl, jl]], rows0, sg0).wait()
        pltpu.make_async_copy(y_hbm.at[src_v.at[curl, jl + 1]], rows1, sg1).wait()

        plsc.subcore_barrier()
        pltpu.sync_copy(
            z_sh.at[pl.ds(sid * rpt, rpt)],
            z_out.at[cid].at[pl.ds(sid * rpt, rpt)],
        )

    return agg_kernel


def _tc1_body(x_ref, w_ref, d0_ref, d1_ref, y_ref, dinv_ref):
    deg = d0_ref[...] + d1_ref[...] + 1.0
    dinv = lax.rsqrt(deg)
    dinv_ref[...] = dinv
    y_ref[...] = jnp.dot(x_ref[...], w_ref[...],
                         preferred_element_type=jnp.float32) * dinv[:, None]


def _tc2_body(z0_ref, z1_ref, y1_ref, dinv_ref, b1_ref, w2_ref,
              emb_ref, y2_ref):
    dinv = dinv_ref[...]
    emb = (z0_ref[...] + z1_ref[...] - y1_ref[...]) * dinv[:, None] \
        + b1_ref[...][None, :]
    emb_ref[...] = emb
    h = jnp.maximum(emb, 0.0)
    y2_ref[...] = jnp.dot(h, w2_ref[...],
                          preferred_element_type=jnp.float32) * dinv[:, None]


def _tc3_body(c_real, z0_ref, z1_ref, y2_ref, dinv_ref, b2_ref, out_ref):
    dinv = dinv_ref[...]
    t = (z0_ref[...] + z1_ref[...] - y2_ref[...]) * dinv[:, None] \
        + b2_ref[...][None, :]
    col = lax.broadcasted_iota(jnp.int32, t.shape, 1)
    t = jnp.where(col < c_real, t, -jnp.inf)
    m = jnp.max(t, axis=1, keepdims=True)
    s = jnp.sum(jnp.exp(t - m), axis=1, keepdims=True)
    out_ref[...] = t - m - jnp.log(s)


def kernel(x, edge_index, W1, b1, W2, b2):
    n, f_in = x.shape
    hid = W1.shape[1]
    c = W2.shape[1]
    e = edge_index.shape[1]

    br = 512
    n_pad = _ru(n + 1, br)
    assert n_pad % (NS * 8) == 0
    ept = _ru(-(-e // NT), IB * CHUNK)
    cpt = ept // CHUNK
    e_pad = ept * NT
    d2 = _ru(c, 64)

    pad_e = e_pad - e
    pad_idx = n + jnp.arange(pad_e, dtype=jnp.int32) % (n_pad - n)
    src = jnp.concatenate(
        [edge_index[0], pad_idx]).reshape(NT * cpt, CHUNK)
    dst = jnp.concatenate(
        [edge_index[1], pad_idx]).reshape(NT * cpt, CHUNK)
    x_pad = jnp.pad(x, ((0, n_pad - n), (0, 0)))
    w2p = jnp.pad(W2, ((0, 0), (0, d2 - c)))
    b2p = jnp.pad(b2, (0, d2 - c))

    grid = n_pad // br
    row_spec = pl.BlockSpec((br, hid), lambda i: (i, 0))
    row_spec2 = pl.BlockSpec((br, d2), lambda i: (i, 0))
    vec_spec = pl.BlockSpec((br,), lambda i: (i,))

    deg3 = _make_deg_kernel(n_pad, cpt)(dst)
    dp0 = deg3[0, :, 0]
    dp1 = deg3[1, :, 0]

    y1, dinv = pl.pallas_call(
        _tc1_body,
        grid=(grid,),
        in_specs=[
            pl.BlockSpec((br, f_in), lambda i: (i, 0)),
            pl.BlockSpec((f_in, hid), lambda i: (0, 0)),
            vec_spec, vec_spec,
        ],
        out_specs=[row_spec, vec_spec],
        out_shape=[
            jax.ShapeDtypeStruct((n_pad, hid), jnp.float32),
            jax.ShapeDtypeStruct((n_pad,), jnp.float32),
        ],
    )(x_pad, W1, dp0, dp1)

    z1 = _make_agg_kernel(n_pad, hid, cpt, True)(y1, src, dst)

    emb, y2 = pl.pallas_call(
        _tc2_body,
        grid=(grid,),
        in_specs=[
            row_spec, row_spec, row_spec, vec_spec,
            pl.BlockSpec((hid,), lambda i: (0,)),
            pl.BlockSpec((hid, d2), lambda i: (0, 0)),
        ],
        out_specs=[row_spec, row_spec2],
        out_shape=[
            jax.ShapeDtypeStruct((n_pad, hid), jnp.float32),
            jax.ShapeDtypeStruct((n_pad, d2), jnp.float32),
        ],
    )(z1[0], z1[1], y1, dinv, b1, w2p)

    z2 = _make_agg_kernel(n_pad, d2, cpt, False)(y2, src, dst)

    out = pl.pallas_call(
        functools.partial(_tc3_body, c),
        grid=(grid,),
        in_specs=[
            row_spec2, row_spec2, row_spec2, vec_spec,
            pl.BlockSpec((d2,), lambda i: (0,)),
        ],
        out_specs=row_spec2,
        out_shape=jax.ShapeDtypeStruct((n_pad, d2), jnp.float32),
    )(z2[0], z2[1], y2, dinv, b2p)

    return (out[:n, :c], emb[:n])


# speedup vs baseline: 28.3237x; 1.0729x over previous
import functools

import jax
import jax.numpy as jnp
from jax import lax
from jax.experimental import pallas as pl
from jax.experimental.pallas import tpu as pltpu
from jax.experimental.pallas import tpu_sc as plsc

NC = 2
NS = 16
NT = NC * NS
CHUNK = 128
IB = 16
BR = 632


def _part(e):
    rows = e // CHUNK
    q8 = rows // 8
    u = q8 // NT
    rem = q8 % NT
    leftover = rows % 8
    max_rows = 8 * (u + 1) + leftover
    rows_pad = rows + 128
    return rows, u, rem, leftover, max_rows, rows_pad


def _tile_range(wid, u, rem, leftover):
    rows_w = 8 * (u + jnp.where(wid < rem, 1, 0)) \
        + jnp.where(wid == NT - 1, leftover, 0)
    base_w = 8 * (u * wid + jnp.minimum(wid, rem))
    return base_w, rows_w


def _make_deg_kernel(n_pad: int, e: int, rows_pad: int, max_rows: int,
                     u: int, rem: int, leftover: int):
    rpt = n_pad // NS
    mesh = plsc.VectorSubcoreMesh(core_axis_name="c", subcore_axis_name="s")

    @functools.partial(
        pl.kernel,
        out_type=jax.ShapeDtypeStruct((NC, n_pad, 16), jnp.float32),
        mesh=mesh,
        compiler_params=pltpu.CompilerParams(use_tc_tiling_on_sc=False),
        scratch_types=[
            pltpu.VMEM((max_rows, CHUNK), jnp.int32),
            pltpu.VMEM((CHUNK, 16), jnp.float32),
            pltpu.VMEM((rpt, 16), jnp.float32),
            pltpu.VMEM_SHARED((n_pad, 16), jnp.float32),
            pltpu.SemaphoreType.DMA,
        ],
    )
    def deg_kernel(dst_hbm, deg_out, dst_v, ones_v, zeros_v, deg_sh, ssd):
        cid = lax.axis_index("c")
        sid = lax.axis_index("s")
        wid = cid * NS + sid
        base_w, rows_w = _tile_range(wid, u, rem, leftover)
        pltpu.sync_copy(dst_hbm.at[pl.ds(base_w, max_rows)], dst_v)

        def fill_ones(i, _):
            ones_v[i] = jnp.full((16,), 1.0, jnp.float32)
            return 0

        lax.fori_loop(0, CHUNK, fill_ones, 0)

        def fill_zeros(i, _):
            zeros_v[i] = jnp.zeros((16,), jnp.float32)
            return 0

        lax.fori_loop(0, rpt, fill_zeros, 0)

        pltpu.sync_copy(zeros_v, deg_sh.at[pl.ds(sid * rpt, rpt)])
        plsc.subcore_barrier()

        lag = 4

        def body(c, _):
            pltpu.async_copy(ones_v, deg_sh.at[dst_v.at[c]], ssd, add=True)

            @pl.when(c >= lag)
            def _():
                pltpu.make_async_copy(
                    ones_v, deg_sh.at[dst_v.at[c - lag]], ssd).wait()

            return 0

        lax.fori_loop(0, rows_w, body, 0)

        def drain(c, _):
            @pl.when(c < jnp.minimum(rows_w, lag))
            def _():
                pltpu.make_async_copy(
                    ones_v, deg_sh.at[dst_v.at[0]], ssd).wait()

            return 0

        lax.fori_loop(0, lag, drain, 0)
        plsc.subcore_barrier()
        pltpu.sync_copy(
            deg_sh.at[pl.ds(sid * rpt, rpt)],
            deg_out.at[cid].at[pl.ds(sid * rpt, rpt)],
        )

    return deg_kernel


def _make_agg_kernel(n_pad: int, d: int, tc_tiling: bool, rows_pad: int,
                     u: int, rem: int, leftover: int):
    rpt = n_pad // NS
    ppb = IB // 2
    mesh = plsc.VectorSubcoreMesh(core_axis_name="c", subcore_axis_name="s")

    @functools.partial(
        pl.kernel,
        out_type=jax.ShapeDtypeStruct((NC, n_pad, d), jnp.float32),
        mesh=mesh,
        compiler_params=pltpu.CompilerParams(use_tc_tiling_on_sc=tc_tiling),
        scratch_types=[
            pltpu.VMEM((2, IB, CHUNK), jnp.int32),
            pltpu.VMEM((2, IB, CHUNK), jnp.int32),
            pltpu.VMEM((CHUNK, d), jnp.float32),
            pltpu.VMEM((CHUNK, d), jnp.float32),
            pltpu.VMEM_SHARED((n_pad, d), jnp.float32),
            pltpu.SemaphoreType.DMA,
            pltpu.SemaphoreType.DMA,
            pltpu.SemaphoreType.DMA,
            pltpu.SemaphoreType.DMA,
            pltpu.SemaphoreType.DMA,
        ],
    )
    def agg_kernel(y_hbm, src_hbm, dst_hbm, z_out,
                   src_v, dst_v, rows0, rows1, z_sh,
                   sg0, sg1, ss0, ss1, si):
        cid = lax.axis_index("c")
        sid = lax.axis_index("s")
        wid = cid * NS + sid
        base_w, rows_w = _tile_range(wid, u, rem, leftover)
        npairs = rows_w // 2

        pltpu.sync_copy(
            y_hbm.at[pl.ds(sid * rpt, rpt)],
            z_sh.at[pl.ds(sid * rpt, rpt)],
        )
        pltpu.sync_copy(src_hbm.at[pl.ds(base_w, IB)], src_v.at[0])
        pltpu.sync_copy(dst_hbm.at[pl.ds(base_w, IB)], dst_v.at[0])
        plsc.subcore_barrier()

        pltpu.async_copy(y_hbm.at[src_v.at[0, 0]], rows0, sg0)
        pltpu.async_copy(y_hbm.at[src_v.at[0, 1]], rows1, sg1)

        def body(p, _):
            t = p // ppb
            cur = lax.rem(t, 2)
            jj = lax.rem(p, ppb) * 2

            @pl.when(jnp.logical_and(jj == 0, (t + 1) * IB < rows_w))
            def _():
                nxt = 1 - cur
                nbase = base_w + (t + 1) * IB
                pltpu.async_copy(src_hbm.at[pl.ds(nbase, IB)],
                                 src_v.at[nxt], si)
                pltpu.async_copy(dst_hbm.at[pl.ds(nbase, IB)],
                                 dst_v.at[nxt], si)

            pltpu.make_async_copy(y_hbm.at[src_v.at[cur, jj]], rows0, sg0).wait()
            pltpu.async_copy(rows0, z_sh.at[dst_v.at[cur, jj]], ss0, add=True)
            pltpu.make_async_copy(y_hbm.at[src_v.at[cur, jj + 1]], rows1, sg1).wait()
            pltpu.async_copy(rows1, z_sh.at[dst_v.at[cur, jj + 1]], ss1, add=True)

            pn = jnp.minimum(p + 1, npairs - 1)
            tn = pn // ppb
            curn = lax.rem(tn, 2)
            jn = lax.rem(pn, ppb) * 2

            @pl.when(tn != t)
            def _():
                nbase = base_w + tn * IB
                pltpu.make_async_copy(src_hbm.at[pl.ds(nbase, IB)],
                                      src_v.at[curn], si).wait()
                pltpu.make_async_copy(dst_hbm.at[pl.ds(nbase, IB)],
                                      dst_v.at[curn], si).wait()

            pltpu.make_async_copy(rows0, z_sh.at[dst_v.at[cur, jj]], ss0).wait()
            pltpu.async_copy(y_hbm.at[src_v.at[curn, jn]], rows0, sg0)
            pltpu.make_async_copy(rows1, z_sh.at[dst_v.at[cur, jj + 1]], ss1).wait()
            pltpu.async_copy(y_hbm.at[src_v.at[curn, jn + 1]], rows1, sg1)
            return 0

        lax.fori_loop(0, npairs, body, 0)
        last = npairs - 1
        tl = last // ppb
        curl = lax.rem(tl, 2)
        jl = lax.rem(last, ppb) * 2
        pltpu.make_async_copy(y_hbm.at[src_v.at[curl, jl]], rows0, sg0).wait()
        pltpu.make_async_copy(y_hbm.at[src_v.at[curl, jl + 1]], rows1, sg1).wait()

        plsc.subcore_barrier()
        pltpu.sync_copy(
            z_sh.at[pl.ds(sid * rpt, rpt)],
            z_out.at[cid].at[pl.ds(sid * rpt, rpt)],
        )

    return agg_kernel


def _dinv_block(d0_ref, d1_ref):
    deg = d0_ref[0][:, 0:1] + d1_ref[0][:, 0:1] + 1.0
    return lax.rsqrt(deg)


def _tc1_body(x_ref, w_ref, d0_ref, d1_ref, y_ref):
    dinv = _dinv_block(d0_ref, d1_ref)
    y_ref[...] = jnp.dot(x_ref[...], w_ref[...],
                         preferred_element_type=jnp.float32) * dinv


def _tc2_body(z0_ref, z1_ref, y1_ref, d0_ref, d1_ref, b1_ref, w2_ref,
              emb_ref, y2_ref):
    dinv = _dinv_block(d0_ref, d1_ref)
    emb = (z0_ref[0] + z1_ref[0] - y1_ref[...]) * dinv + b1_ref[...][None, :]
    emb_ref[...] = emb
    h = jnp.maximum(emb, 0.0)
    y2_ref[...] = jnp.dot(h, w2_ref[...],
                          preferred_element_type=jnp.float32) * dinv


def _tc3_body(c_real, z0_ref, z1_ref, y2_ref, d0_ref, d1_ref, b2_ref, out_ref):
    dinv = _dinv_block(d0_ref, d1_ref)
    t = (z0_ref[0] + z1_ref[0] - y2_ref[...]) * dinv + b2_ref[...][None, :]
    col = lax.broadcasted_iota(jnp.int32, t.shape, 1)
    t = jnp.where(col < c_real, t, -jnp.inf)
    m = jnp.max(t, axis=1, keepdims=True)
    s = jnp.sum(jnp.exp(t - m), axis=1, keepdims=True)
    out_ref[...] = (t - m - jnp.log(s))[:, :c_real]


def kernel(x, edge_index, W1, b1, W2, b2):
    n, f_in = x.shape
    hid = W1.shape[1]
    c = W2.shape[1]
    e = edge_index.shape[1]
    assert e % CHUNK == 0

    n_pad = NS * BR
    assert n_pad >= n and n_pad % (NS * 8) == 0
    d2 = 64
    assert c <= d2

    rows, u, rem, leftover, max_rows, rows_pad = _part(e)

    src = jnp.pad(edge_index[0], (0, rows_pad * CHUNK - e)).reshape(
        rows_pad, CHUNK)
    dst = jnp.pad(edge_index[1], (0, rows_pad * CHUNK - e)).reshape(
        rows_pad, CHUNK)
    w2p = jnp.pad(W2, ((0, 0), (0, d2 - c)))
    b2p = jnp.pad(b2, (0, d2 - c))

    grid = n_pad // BR
    row_spec = pl.BlockSpec((BR, hid), lambda i: (i, 0))
    row_spec2 = pl.BlockSpec((BR, d2), lambda i: (i, 0))
    deg_spec0 = pl.BlockSpec((1, BR, 16), lambda i: (0, i, 0))
    deg_spec1 = pl.BlockSpec((1, BR, 16), lambda i: (1, i, 0))

    deg3 = _make_deg_kernel(n_pad, e, rows_pad, max_rows, u, rem, leftover)(dst)

    y1 = pl.pallas_call(
        _tc1_body,
        grid=(grid,),
        in_specs=[
            pl.BlockSpec((BR, f_in), lambda i: (i, 0)),
            pl.BlockSpec((f_in, hid), lambda i: (0, 0)),
            deg_spec0, deg_spec1,
        ],
        out_specs=row_spec,
        out_shape=jax.ShapeDtypeStruct((n_pad, hid), jnp.float32),
    )(x, W1, deg3, deg3)

    z1 = _make_agg_kernel(n_pad, hid, True, rows_pad, u, rem, leftover)(
        y1, src, dst)

    emb, y2 = pl.pallas_call(
        _tc2_body,
        grid=(grid,),
        in_specs=[
            pl.BlockSpec((1, BR, hid), lambda i: (0, i, 0)),
            pl.BlockSpec((1, BR, hid), lambda i: (1, i, 0)),
            row_spec, deg_spec0, deg_spec1,
            pl.BlockSpec((hid,), lambda i: (0,)),
            pl.BlockSpec((hid, d2), lambda i: (0, 0)),
        ],
        out_specs=[row_spec, row_spec2],
        out_shape=[
            jax.ShapeDtypeStruct((n, hid), jnp.float32),
            jax.ShapeDtypeStruct((n_pad, d2), jnp.float32),
        ],
    )(z1, z1, y1, deg3, deg3, b1, w2p)

    z2 = _make_agg_kernel(n_pad, d2, False, rows_pad, u, rem, leftover)(
        y2, src, dst)

    out = pl.pallas_call(
        functools.partial(_tc3_body, c),
        grid=(grid,),
        in_specs=[
            pl.BlockSpec((1, BR, d2), lambda i: (0, i, 0)),
            pl.BlockSpec((1, BR, d2), lambda i: (1, i, 0)),
            row_spec2, deg_spec0, deg_spec1,
            pl.BlockSpec((d2,), lambda i: (0,)),
        ],
        out_specs=pl.BlockSpec((BR, c), lambda i: (i, 0)),
        out_shape=jax.ShapeDtypeStruct((n, c), jnp.float32),
    )(z2, z2, y2, deg3, deg3, b2p)

    return (out, emb)


# speedup vs baseline: 30.0514x; 1.0610x over previous
import functools

import jax
import jax.numpy as jnp
from jax import lax
from jax.experimental import pallas as pl
from jax.experimental.pallas import tpu as pltpu
from jax.experimental.pallas import tpu_sc as plsc

NC = 2
NS = 16
NT = NC * NS
CHUNK = 128
IB = 16
BR = 512


def _part(e):
    rows = e // CHUNK
    q8 = rows // 8
    u = q8 // NT
    rem = q8 % NT
    leftover = rows % 8
    max_rows = 8 * (u + 1) + leftover
    return rows, u, rem, leftover, max_rows


def _tile_range(wid, u, rem, leftover):
    rows_w = 8 * (u + jnp.where(wid < rem, 1, 0)) \
        + jnp.where(wid == NT - 1, leftover, 0)
    base_w = 8 * (u * wid + jnp.minimum(wid, rem))
    return base_w, rows_w


def _make_deg_kernel(n_deg: int, rows: int, max_rows: int,
                     u: int, rem: int, leftover: int):
    rpt = n_deg // NS
    mesh = plsc.VectorSubcoreMesh(core_axis_name="c", subcore_axis_name="s")

    @functools.partial(
        pl.kernel,
        out_type=[
            jax.ShapeDtypeStruct((n_deg,), jnp.float32),
            jax.ShapeDtypeStruct((n_deg,), jnp.float32),
        ],
        mesh=mesh,
        compiler_params=pltpu.CompilerParams(use_tc_tiling_on_sc=False,
                                             needs_layout_passes=False),
        scratch_types=[
            pltpu.VMEM((max_rows * CHUNK,), jnp.int32),
            pltpu.VMEM((n_deg,), jnp.float32),
            pltpu.VMEM((rpt,), jnp.float32),
            pltpu.VMEM((rpt,), jnp.float32),
            pltpu.VMEM_SHARED((NS * n_deg,), jnp.float32),
        ],
    )
    def deg_kernel(edge_hbm, deg0, deg1, dst_v, hist, acc, tmp, sh):
        cid = lax.axis_index("c")
        sid = lax.axis_index("s")
        wid = cid * NS + sid
        base_w, rows_w = _tile_range(wid, u, rem, leftover)
        pltpu.sync_copy(
            edge_hbm.at[1].at[pl.ds(base_w * CHUNK, max_rows * CHUNK)], dst_v)

        def fill_zeros(i, _):
            hist[pl.ds(i * 16, 16)] = jnp.zeros((16,), jnp.float32)
            return 0

        lax.fori_loop(0, n_deg // 16, fill_zeros, 0)

        ones16 = jnp.full((16,), 1.0, jnp.float32)

        def body(c, _):
            r = c * CHUNK
            for l in range(CHUNK // 16):
                idx = dst_v[pl.ds(r + l * 16, 16)]
                plsc.addupdate_scatter(hist, [idx], ones16)
            return 0

        lax.fori_loop(0, rows_w, body, 0)
        pltpu.sync_copy(hist, sh.at[pl.ds(sid * n_deg, n_deg)])
        plsc.subcore_barrier()

        pltpu.sync_copy(sh.at[pl.ds(sid * rpt, rpt)], acc)

        def red(r_, _):
            pltpu.sync_copy(sh.at[pl.ds(r_ * n_deg + sid * rpt, rpt)], tmp)

            def add(k, _):
                acc[pl.ds(k * 16, 16)] = acc[pl.ds(k * 16, 16)] \
                    + tmp[pl.ds(k * 16, 16)]
                return 0

            lax.fori_loop(0, rpt // 16, add, 0)
            return 0

        lax.fori_loop(1, NS, red, 0)

        @pl.when(cid == 0)
        def _():
            pltpu.sync_copy(acc, deg0.at[pl.ds(sid * rpt, rpt)])

        @pl.when(cid == 1)
        def _():
            pltpu.sync_copy(acc, deg1.at[pl.ds(sid * rpt, rpt)])

    return deg_kernel


def _make_agg_kernel(n_pad: int, d: int, tc_tiling: bool, rows: int,
                     u: int, rem: int, leftover: int):
    rpt = n_pad // NS
    ppb = IB // 2
    mesh = plsc.VectorSubcoreMesh(core_axis_name="c", subcore_axis_name="s")

    @functools.partial(
        pl.kernel,
        out_type=jax.ShapeDtypeStruct((NC, n_pad, d), jnp.float32),
        mesh=mesh,
        compiler_params=pltpu.CompilerParams(use_tc_tiling_on_sc=tc_tiling),
        scratch_types=[
            pltpu.VMEM((2, IB, CHUNK), jnp.int32),
            pltpu.VMEM((2, IB, CHUNK), jnp.int32),
            pltpu.VMEM((CHUNK, d), jnp.float32),
            pltpu.VMEM((CHUNK, d), jnp.float32),
            pltpu.VMEM_SHARED((n_pad, d), jnp.float32),
            pltpu.SemaphoreType.DMA,
            pltpu.SemaphoreType.DMA,
            pltpu.SemaphoreType.DMA,
            pltpu.SemaphoreType.DMA,
            pltpu.SemaphoreType.DMA,
        ],
    )
    def agg_kernel(y_hbm, edge_hbm, z_out,
                   src_v, dst_v, rows0, rows1, z_sh,
                   sg0, sg1, ss0, ss1, si):
        cid = lax.axis_index("c")
        sid = lax.axis_index("s")
        wid = cid * NS + sid
        base_w, rows_w = _tile_range(wid, u, rem, leftover)
        npairs = rows_w // 2
        src_hbm = edge_hbm.at[0]
        dst_hbm = edge_hbm.at[1]

        pltpu.sync_copy(
            y_hbm.at[pl.ds(sid * rpt, rpt)],
            z_sh.at[pl.ds(sid * rpt, rpt)],
        )
        pltpu.sync_copy(src_hbm.at[pl.ds(base_w, IB)], src_v.at[0])
        pltpu.sync_copy(dst_hbm.at[pl.ds(base_w, IB)], dst_v.at[0])
        plsc.subcore_barrier()

        pltpu.async_copy(y_hbm.at[src_v.at[0, 0]], rows0, sg0)
        pltpu.async_copy(y_hbm.at[src_v.at[0, 1]], rows1, sg1)

        def body(p, _):
            t = p // ppb
            cur = lax.rem(t, 2)
            jj = lax.rem(p, ppb) * 2
            nb = base_w + (t + 1) * IB

            @pl.when(jnp.logical_and(jj == 0, (t + 1) * IB < rows_w))
            def _():
                nxt = 1 - cur
                pltpu.async_copy(src_hbm.at[pl.ds(nb, IB)],
                                 src_v.at[nxt], si)
                pltpu.async_copy(dst_hbm.at[pl.ds(nb, IB)],
                                 dst_v.at[nxt], si)

            pltpu.make_async_copy(y_hbm.at[src_v.at[cur, jj]], rows0, sg0).wait()
            pltpu.async_copy(rows0, z_sh.at[dst_v.at[cur, jj]], ss0, add=True)
            pltpu.make_async_copy(y_hbm.at[src_v.at[cur, jj + 1]], rows1, sg1).wait()
            pltpu.async_copy(rows1, z_sh.at[dst_v.at[cur, jj + 1]], ss1, add=True)

            pn = jnp.minimum(p + 1, npairs - 1)
            tn = pn // ppb
            curn = lax.rem(tn, 2)
            jn = lax.rem(pn, ppb) * 2

            @pl.when(tn != t)
            def _():
                pltpu.make_async_copy(src_hbm.at[pl.ds(nb, IB)],
                                      src_v.at[curn], si).wait()
                pltpu.make_async_copy(dst_hbm.at[pl.ds(nb, IB)],
                                      dst_v.at[curn], si).wait()

            pltpu.make_async_copy(rows0, z_sh.at[dst_v.at[cur, jj]], ss0).wait()
            pltpu.async_copy(y_hbm.at[src_v.at[curn, jn]], rows0, sg0)
            pltpu.make_async_copy(rows1, z_sh.at[dst_v.at[cur, jj + 1]], ss1).wait()
            pltpu.async_copy(y_hbm.at[src_v.at[curn, jn + 1]], rows1, sg1)
            return 0

        lax.fori_loop(0, npairs, body, 0)
        last = npairs - 1
        tl = last // ppb
        curl = lax.rem(tl, 2)
        jl = lax.rem(last, ppb) * 2
        pltpu.make_async_copy(y_hbm.at[src_v.at[curl, jl]], rows0, sg0).wait()
        pltpu.make_async_copy(y_hbm.at[src_v.at[curl, jl + 1]], rows1, sg1).wait()

        plsc.subcore_barrier()
        pltpu.sync_copy(
            z_sh.at[pl.ds(sid * rpt, rpt)],
            z_out.at[cid].at[pl.ds(sid * rpt, rpt)],
        )

    return agg_kernel


def _dinv_block(d0_ref, d1_ref):
    i = pl.program_id(0)
    deg = d0_ref[pl.ds(i * BR, BR)] + d1_ref[pl.ds(i * BR, BR)] + 1.0
    return lax.rsqrt(deg)[:, None]


def _tc1_body(x_ref, w_ref, d0_ref, d1_ref, y_ref):
    dinv = _dinv_block(d0_ref, d1_ref)
    y_ref[...] = jnp.dot(x_ref[...], w_ref[...],
                         preferred_element_type=jnp.float32) * dinv


def _tc2_body(z0_ref, z1_ref, y1_ref, d0_ref, d1_ref, b1_ref, w2_ref,
              emb_ref, y2_ref):
    dinv = _dinv_block(d0_ref, d1_ref)
    emb = (z0_ref[0] + z1_ref[0] - y1_ref[...]) * dinv + b1_ref[...][None, :]
    emb_ref[...] = emb
    h = jnp.maximum(emb, 0.0)
    y2_ref[...] = jnp.dot(h, w2_ref[...],
                          preferred_element_type=jnp.float32) * dinv


def _tc3_body(c_real, z0_ref, z1_ref, y2_ref, d0_ref, d1_ref, b2_ref, out_ref):
    dinv = _dinv_block(d0_ref, d1_ref)
    t = (z0_ref[0] + z1_ref[0] - y2_ref[...]) * dinv + b2_ref[...][None, :]
    col = lax.broadcasted_iota(jnp.int32, t.shape, 1)
    t = jnp.where(col < c_real, t, -jnp.inf)
    m = jnp.max(t, axis=1, keepdims=True)
    s = jnp.sum(jnp.exp(t - m), axis=1, keepdims=True)
    out_ref[...] = (t - m - jnp.log(s))[:, :c_real]


def kernel(x, edge_index, W1, b1, W2, b2):
    n, f_in = x.shape
    hid = W1.shape[1]
    c = W2.shape[1]
    e = edge_index.shape[1]
    assert e % CHUNK == 0

    n_pad = 20 * BR
    assert n_pad >= n and n_pad % (NS * 8) == 0
    n_deg = n_pad
    assert n_deg % (NS * 16) == 0
    d2 = 64
    assert c <= d2

    rows, u, rem, leftover, max_rows = _part(e)

    rows_pad = rows + 12
    edge_pad = jnp.pad(edge_index, ((0, 0), (0, (rows_pad - rows) * CHUNK)))
    edge3 = edge_pad.reshape(2, rows_pad, CHUNK)
    w2p = jnp.pad(W2, ((0, 0), (0, d2 - c)))
    b2p = jnp.pad(b2, (0, d2 - c))

    grid = n_pad // BR
    row_spec = pl.BlockSpec((BR, hid), lambda i: (i, 0))
    row_spec2 = pl.BlockSpec((BR, d2), lambda i: (i, 0))
    deg_spec = pl.BlockSpec((n_deg,), lambda i: (0,))

    deg0, deg1 = _make_deg_kernel(n_deg, rows, max_rows, u, rem, leftover)(
        edge_pad)

    y1 = pl.pallas_call(
        _tc1_body,
        grid=(grid,),
        in_specs=[
            pl.BlockSpec((BR, f_in), lambda i: (i, 0)),
            pl.BlockSpec((f_in, hid), lambda i: (0, 0)),
            deg_spec, deg_spec,
        ],
        out_specs=row_spec,
        out_shape=jax.ShapeDtypeStruct((n_pad, hid), jnp.float32),
    )(x, W1, deg0, deg1)

    z1 = _make_agg_kernel(n_pad, hid, True, rows, u, rem, leftover)(y1, edge3)

    emb, y2 = pl.pallas_call(
        _tc2_body,
        grid=(grid,),
        in_specs=[
            pl.BlockSpec((1, BR, hid), lambda i: (0, i, 0)),
            pl.BlockSpec((1, BR, hid), lambda i: (1, i, 0)),
            row_spec, deg_spec, deg_spec,
            pl.BlockSpec((hid,), lambda i: (0,)),
            pl.BlockSpec((hid, d2), lambda i: (0, 0)),
        ],
        out_specs=[row_spec, row_spec2],
        out_shape=[
            jax.ShapeDtypeStruct((n, hid), jnp.float32),
            jax.ShapeDtypeStruct((n_pad, d2), jnp.float32),
        ],
    )(z1, z1, y1, deg0, deg1, b1, w2p)

    z2 = _make_agg_kernel(n_pad, d2, False, rows, u, rem, leftover)(y2, edge3)

    out = pl.pallas_call(
        functools.partial(_tc3_body, c),
        grid=(grid,),
        in_specs=[
            pl.BlockSpec((1, BR, d2), lambda i: (0, i, 0)),
            pl.BlockSpec((1, BR, d2), lambda i: (1, i, 0)),
            row_spec2, deg_spec, deg_spec,
            pl.BlockSpec((d2,), lambda i: (0,)),
        ],
        out_specs=pl.BlockSpec((BR, c), lambda i: (i, 0)),
        out_shape=jax.ShapeDtypeStruct((n, c), jnp.float32),
    )(z2, z2, y2, deg0, deg1, b2p)

    return (out, emb)
